# baseline (speedup 1.0000x reference)
import jax
import jax.numpy as jnp
from jax import lax
from jax.experimental import pallas as pl
from jax.experimental.pallas import tpu as pltpu
from jax.experimental.pallas import tpu_sc as plsc

N_NODES = 10000
N_EDGES = 320000
FEAT = 128
LANES = 16
FB = FEAT // LANES
WIN = 128
NWIN_MAX = N_EDGES // WIN
ROWS_PER = 320
RP_STAGE = 337
RP_LIMIT = 9664
RP_PAD = RP_STAGE + 2 * LANES
NV_CHUNK = 640
ORING = 64
PAD_ROW = N_NODES
OUT_PAD = N_NODES + 16


def _tec_body(rp_hbm, ci_hbm, es_hbm, nv_hbm, out_hbm,
              rp_v, sbuf, cbuf, gbuf, oring, pbuf, xbuf, sem_sc, sem_g, sem_o):
    wid = lax.axis_index("s") * 2 + lax.axis_index("c")
    r0 = wid * ROWS_PER
    nrows = jnp.minimum(ROWS_PER, N_NODES - r0)


    rbase = pl.multiple_of(jnp.minimum(r0 - lax.rem(r0, 8), RP_LIMIT), 8)
    pltpu.sync_copy(rp_hbm.at[pl.ds(rbase, RP_STAGE)],
                    rp_v.at[pl.ds(0, RP_STAGE)])
    off = r0 - rbase

    def rp_at(i):
        return rp_v[pl.ds(i, LANES)][0]

    s0 = rp_at(off)
    s1 = rp_at(off + nrows)

    zero16 = jnp.zeros((LANES,), jnp.float32)

    iota = lax.iota(jnp.int32, LANES)
    nrvec = lax.div(nrows + (LANES - 1), LANES)
    _gdn = lax.GatherDimensionNumbers(
        offset_dims=(), collapsed_slice_dims=(0,), start_index_map=(0,))
    lane_splats = [jnp.full((LANES, 1), l, jnp.int32) for l in range(LANES)]

    def bcast(w, l):
        return lax.gather(w, lane_splats[l], _gdn, (1,),
                          mode=lax.GatherScatterMode.PROMISE_IN_BOUNDS)

    def count_ends(whi):
        def cbody(k, uv):
            idx = off + 1 + k * LANES
            ends = rp_v[pl.ds(idx, LANES)]
            m = (ends <= whi) & (k * LANES + iota < nrows)
            return uv + jnp.where(m, 1.0, 0.0)

        uv = lax.fori_loop(0, nrvec, cbody, jnp.zeros((LANES,), jnp.float32))
        return jnp.sum(uv).astype(jnp.int32)

    j0 = lax.div(s0, WIN)
    j1 = lax.div(s1 + (WIN - 1), WIN)

    def wdma(j):
        return pl.multiple_of(
            jnp.minimum(j, NWIN_MAX - 1) * WIN, WIN)

    def issue_sc(j, p):
        base = wdma(j)
        pltpu.make_async_copy(es_hbm.at[pl.ds(base, WIN)],
                              sbuf.at[p], sem_sc).start()
        pltpu.make_async_copy(ci_hbm.at[pl.ds(base, WIN)],
                              cbuf.at[p], sem_sc).start()

    def wait_sc(p):
        pltpu.make_async_copy(es_hbm.at[pl.ds(0, WIN)],
                              sbuf.at[p], sem_sc).wait()
        pltpu.make_async_copy(ci_hbm.at[pl.ds(0, WIN)],
                              cbuf.at[p], sem_sc).wait()

    def clamp_issue_gather(q, p):
        for k in range(WIN // LANES):
            sl = pl.ds(k * LANES, LANES)
            cbuf[q, sl] = jnp.clip(cbuf[q, sl], 0, N_NODES - 1)
        pltpu.make_async_copy(nv_hbm.at[cbuf.at[q]],
                              gbuf.at[p], sem_g).start()

    def wait_gather(p):
        pltpu.make_async_copy(nv_hbm.at[cbuf.at[0]],
                              gbuf.at[p], sem_g).wait()

    def bcast_dyn(v, l):
        return lax.gather(v, jnp.full((LANES, 1), 1, jnp.int32) * l, _gdn,
                          (1,), mode=lax.GatherScatterMode.PROMISE_IN_BOUNDS)

    def compute(j, p, q, d, r_in, denv, accs):
        wbase = pl.multiple_of(j * WIN, WIN)
        wlo = jnp.maximum(s0, wbase)
        whi = jnp.minimum(s1, wbase + WIN)
        cnt = count_ends(whi) - r_in

        run = [zero16] * FB
        basev = zero16
        for sslot in range(WIN // LANES):
            lbase = sslot * LANES
            sv = sbuf[q, pl.ds(lbase, LANES)]
            gidx = wbase + lbase + iota
            m = (gidx >= s0) & (gidx < s1)
            w = jnp.where(m, jnp.exp(sv), 0.0)
            incl = plsc.cumsum(w)
            xbuf[sslot, pl.ds(0, LANES)] = basev + incl - w
            basev = basev + bcast(incl, LANES - 1)
            for l in range(LANES):
                wl = bcast(w, l)
                gq = lbase + l
                for k in range(FB):
                    sl = pl.ds(k * LANES, LANES)
                    pbuf[gq, sl] = run[k]
                    run[k] = run[k] + wl * gbuf[p, gq, sl]
        xbuf[WIN // LANES, pl.ds(0, LANES)] = basev
        for k in range(FB):
            pbuf[WIN, pl.ds(k * LANES, LANES)] = run[k]

        def wsum_at(i):
            sv = xbuf[lax.div(i, LANES), pl.ds(0, LANES)]
            return bcast_dyn(sv, lax.rem(i, LANES))

        def row_body(t, rcar):
            denv, accs = rcar[0], list(rcar[1:])
            fin = t < cnt
            rr = jnp.minimum(r_in + t, nrows - 1)
            r_end = rp_at(off + rr + 1)
            a = jnp.maximum(rp_at(off + rr), wlo)
            b = jnp.minimum(r_end, whi)
            has = b > a
            a_loc = jnp.where(has, a - wbase, 0)
            b_loc = jnp.where(has, b - wbase, 0)
            denv = denv + wsum_at(b_loc) - wsum_at(a_loc)
            for k in range(FB):
                sl = pl.ds(k * LANES, LANES)
                accs[k] = accs[k] + (pbuf[b_loc, sl] - pbuf[a_loc, sl])

            scale = jnp.where(denv > 0.0, 1.0 / denv, 0.0)
            slot = lax.rem(r_in + t + d, ORING)
            for k in range(FB):
                sl = pl.ds(k * LANES, LANES)
                oring[slot, sl] = accs[k] * scale
                accs[k] = jnp.where(fin, zero16, accs[k])
            denv = jnp.where(fin, zero16, denv)
            tgt = jnp.where(fin, r0 + rr, PAD_ROW)
            pltpu.make_async_copy(
                oring.at[slot],
                out_hbm.at[pl.ds(pl.multiple_of(tgt * FEAT, 8), FEAT)],
                sem_o).start()
            return (denv, *accs)

        denv, *accs = lax.fori_loop(0, cnt + 1, row_body, (denv, *accs))
        return r_in + cnt, denv, accs

    issue_sc(j0, 0)
    wait_sc(0)
    clamp_issue_gather(0, 0)
    issue_sc(j0 + 1, 1)

    def win_body(j, wcar):
        r_in, denv, accs = wcar[0], wcar[1], list(wcar[2:])
        d = j - j0
        p = lax.rem(d, 2)
        pn = 1 - p
        q = lax.rem(d, 3)
        q1 = lax.rem(d + 1, 3)
        q2 = lax.rem(d + 2, 3)
        wait_gather(p)
        wait_sc(q1)
        clamp_issue_gather(q1, pn)
        issue_sc(j + 2, q2)
        r_out, denv, accs = compute(j, p, q, d, r_in, denv, accs)
        return (r_out, denv, *accs)

    wcar0 = (jnp.int32(0), zero16, *([zero16] * FB))
    r_mid = lax.fori_loop(j0, j1, win_body, wcar0)[0]

    wait_gather(lax.rem(j1 - j0, 2))
    wait_sc(lax.rem(j1 + 1 - j0, 3))

    nwin = j1 - j0

    def fin_body(r, _):
        slot = lax.rem(r + nwin, ORING)
        for k in range(FB):
            oring[slot, pl.ds(k * LANES, LANES)] = zero16
        pltpu.make_async_copy(
            oring.at[slot],
            out_hbm.at[pl.ds(pl.multiple_of((r0 + r) * FEAT, 8), FEAT)],
            sem_o).start()
        return 0

    lax.fori_loop(r_mid, nrows, fin_body, 0)

    def drain_body(i, _):
        pltpu.make_async_copy(oring.at[0],
                              out_hbm.at[pl.ds(0, FEAT)], sem_o).wait()
        return 0

    lax.fori_loop(0, nrows + nwin, drain_body, 0)


def kernel(row_ptr, col_idx, edge_scores, node_value):
    mesh = plsc.VectorSubcoreMesh(core_axis_name="c", subcore_axis_name="s")
    run = pl.kernel(
        _tec_body,
        out_type=jax.ShapeDtypeStruct((OUT_PAD * FEAT,), jnp.float32),
        mesh=mesh,
        scratch_types=[
            pltpu.VMEM((RP_PAD,), jnp.int32),
            pltpu.VMEM((3, WIN), jnp.float32),
            pltpu.VMEM((3, WIN), jnp.int32),
            pltpu.VMEM((2, WIN, FEAT), jnp.float32),
            pltpu.VMEM((ORING, FEAT), jnp.float32),
            pltpu.VMEM((WIN + 1, FEAT), jnp.float32),
            pltpu.VMEM((WIN // LANES + 1, LANES), jnp.float32),
            pltpu.SemaphoreType.DMA,
            pltpu.SemaphoreType.DMA,
            pltpu.SemaphoreType.DMA,
        ],
        compiler_params=pltpu.CompilerParams(needs_layout_passes=False),
    )
    flat = run(row_ptr.astype(jnp.int32), col_idx.astype(jnp.int32),
               edge_scores, node_value)
    return flat[:N_NODES * FEAT].reshape(N_NODES, FEAT)


# speedup vs baseline: 1.8123x; 1.8123x over previous
import jax
import jax.numpy as jnp
from jax import lax
from jax.experimental import pallas as pl
from jax.experimental.pallas import tpu as pltpu
from jax.experimental.pallas import tpu_sc as plsc

N_NODES = 10000
N_EDGES = 320000
FEAT = 128
LANES = 16
FB = FEAT // LANES
WIN = 128
NWIN_MAX = N_EDGES // WIN
ROWS_PER = 320
RP_STAGE = 337
RP_LIMIT = 9664
RP_PAD = RP_STAGE + 2 * LANES


def _tec_body(rp_hbm, ci_hbm, es_hbm, nv_hbm, out_hbm,
              rp_v, sbuf, cbuf, gbuf, obuf, sem_sc, sem_g):
    wid = lax.axis_index("s") * 2 + lax.axis_index("c")
    r0 = wid * ROWS_PER
    nrows = jnp.minimum(ROWS_PER, N_NODES - r0)

    rbase = pl.multiple_of(jnp.minimum(r0 - lax.rem(r0, 8), RP_LIMIT), 8)
    pltpu.sync_copy(rp_hbm.at[pl.ds(rbase, RP_STAGE)],
                    rp_v.at[pl.ds(0, RP_STAGE)])
    off = r0 - rbase

    def rp_at(i):
        return rp_v[pl.ds(i, LANES)][0]

    s0 = rp_at(off)
    s1 = rp_at(off + nrows)

    zero16 = jnp.zeros((LANES,), jnp.float32)
    for _p in range(2):
        for _r in range(LANES):
            for _k in range(FB):
                gbuf[_p, WIN + _r, pl.ds(_k * LANES, LANES)] = zero16

    iota = lax.iota(jnp.int32, LANES)
    nrvec = lax.div(nrows + (LANES - 1), LANES)
    _gdn = lax.GatherDimensionNumbers(
        offset_dims=(), collapsed_slice_dims=(0,), start_index_map=(0,))
    lane_splats = [jnp.full((LANES, 1), l, jnp.int32) for l in range(LANES)]

    def bcast(w, l):
        return lax.gather(w, lane_splats[l], _gdn, (1,),
                          mode=lax.GatherScatterMode.PROMISE_IN_BOUNDS)

    def count_ends(whi):
        def cbody(k, uv):
            idx = off + 1 + k * LANES
            ends = rp_v[pl.ds(idx, LANES)]
            m = (ends <= whi) & (k * LANES + iota < nrows)
            return uv + jnp.where(m, 1.0, 0.0)

        uv = lax.fori_loop(0, nrvec, cbody, jnp.zeros((LANES,), jnp.float32))
        return jnp.sum(uv).astype(jnp.int32)

    j0 = lax.div(s0, WIN)
    j1 = lax.div(s1 + (WIN - 1), WIN)

    def wdma(j):
        return pl.multiple_of(
            jnp.minimum(j, NWIN_MAX - 1) * WIN, WIN)

    def issue_sc(j, p):
        base = wdma(j)
        pltpu.make_async_copy(es_hbm.at[pl.ds(base, WIN)],
                              sbuf.at[p, pl.ds(0, WIN)], sem_sc).start()
        pltpu.make_async_copy(ci_hbm.at[pl.ds(base, WIN)],
                              cbuf.at[p], sem_sc).start()

    def wait_sc(p):
        pltpu.make_async_copy(es_hbm.at[pl.ds(0, WIN)],
                              sbuf.at[p, pl.ds(0, WIN)], sem_sc).wait()
        pltpu.make_async_copy(ci_hbm.at[pl.ds(0, WIN)],
                              cbuf.at[p], sem_sc).wait()

    def clamp_issue_gather(q, p):
        for k in range(WIN // LANES):
            sl = pl.ds(k * LANES, LANES)
            cbuf[q, sl] = jnp.clip(cbuf[q, sl], 0, N_NODES - 1)
        pltpu.make_async_copy(nv_hbm.at[cbuf.at[q]],
                              gbuf.at[p, pl.ds(0, WIN)], sem_g).start()

    def wait_gather(p):
        pltpu.make_async_copy(nv_hbm.at[cbuf.at[0]],
                              gbuf.at[p, pl.ds(0, WIN)], sem_g).wait()

    def compute(j, p, q, r_in, denv, accs):
        wbase = pl.multiple_of(j * WIN, WIN)
        wlo = jnp.maximum(s0, wbase)
        whi = jnp.minimum(s1, wbase + WIN)
        cnt = count_ends(whi) - r_in

        def row_body(t, rcar):
            denv, accs = rcar[0], list(rcar[1:])
            fin = t < cnt
            rr = jnp.minimum(r_in + t, nrows - 1)
            r_end = rp_at(off + rr + 1)
            a = jnp.maximum(rp_at(off + rr), wlo)
            b = jnp.minimum(r_end, whi)

            a_loc = a - wbase

            def slot_body(t, scar):
                denv, accs = scar[0], list(scar[1:])
                lbase = a_loc + t * LANES
                sv = sbuf[q, pl.ds(lbase, LANES)]
                gidx = wbase + lbase + iota
                m = gidx < b
                w = jnp.where(m, jnp.exp(sv), 0.0)
                denv = denv + w
                for l in range(LANES):
                    wl = bcast(w, l)
                    gq = lbase + l
                    for k in range(FB):
                        sl = pl.ds(k * LANES, LANES)
                        accs[k] = accs[k] + wl * gbuf[p, gq, sl]
                return (denv, *accs)

            has = b > a
            nvec = lax.div(b - a + (LANES - 1), LANES)
            denv, *accs = lax.fori_loop(
                0, jnp.where(has, nvec, 0), slot_body, (denv, *accs))

            den = jnp.sum(denv)
            dbv = jnp.broadcast_to(den, (LANES,))
            scale = jnp.where(dbv > 0.0, 1.0 / dbv, 0.0)
            rw = jnp.where(fin, rr, ROWS_PER)
            for k in range(FB):
                sl = pl.ds(k * LANES, LANES)
                obuf[rw, sl] = accs[k] * scale
                accs[k] = jnp.where(fin, zero16, accs[k])
            denv = jnp.where(fin, zero16, denv)
            return (denv, *accs)

        denv, *accs = lax.fori_loop(0, cnt + 1, row_body, (denv, *accs))
        return r_in + cnt, denv, accs

    issue_sc(j0, 0)
    wait_sc(0)
    clamp_issue_gather(0, 0)
    issue_sc(j0 + 1, 1)

    def win_body(j, wcar):
        r_in, denv, accs = wcar[0], wcar[1], list(wcar[2:])
        d = j - j0
        p = lax.rem(d, 2)
        pn = 1 - p
        q = lax.rem(d, 3)
        q1 = lax.rem(d + 1, 3)
        q2 = lax.rem(d + 2, 3)
        wait_gather(p)
        wait_sc(q1)
        clamp_issue_gather(q1, pn)
        issue_sc(j + 2, q2)
        r_out, denv, accs = compute(j, p, q, r_in, denv, accs)
        return (r_out, denv, *accs)

    wcar0 = (jnp.int32(0), zero16, *([zero16] * FB))
    r_mid = lax.fori_loop(j0, j1, win_body, wcar0)[0]

    wait_gather(lax.rem(j1 - j0, 2))
    wait_sc(lax.rem(j1 + 1 - j0, 3))

    def fin_body(r, _):
        for k in range(FB):
            obuf[r, pl.ds(k * LANES, LANES)] = zero16
        return 0

    lax.fori_loop(r_mid, nrows, fin_body, 0)

    ngroups = lax.div(nrows, 8)

    def out_body(g, _):
        dst = pl.multiple_of(r0 + g * 8, 8)
        pltpu.sync_copy(obuf.at[pl.ds(g * 8, 8), :],
                        out_hbm.at[pl.ds(dst, 8), :])
        return 0

    lax.fori_loop(0, ngroups, out_body, 0)


def kernel(row_ptr, col_idx, edge_scores, node_value):
    mesh = plsc.VectorSubcoreMesh(core_axis_name="c", subcore_axis_name="s")
    run = pl.kernel(
        _tec_body,
        out_type=jax.ShapeDtypeStruct((N_NODES, FEAT), jnp.float32),
        mesh=mesh,
        scratch_types=[
            pltpu.VMEM((RP_PAD,), jnp.int32),
            pltpu.VMEM((3, WIN + LANES), jnp.float32),
            pltpu.VMEM((3, WIN), jnp.int32),
            pltpu.VMEM((2, WIN + LANES, FEAT), jnp.float32),
            pltpu.VMEM((ROWS_PER + 1, FEAT), jnp.float32),
            pltpu.SemaphoreType.DMA,
            pltpu.SemaphoreType.DMA,
        ],
        compiler_params=pltpu.CompilerParams(needs_layout_passes=False),
    )
    return run(row_ptr.astype(jnp.int32), col_idx.astype(jnp.int32),
               edge_scores, node_value)
